# Initial kernel scaffold; baseline (speedup 1.0000x reference)
import jax
import jax.numpy as jnp
from jax.experimental import pallas as pl


def kernel(activ_last_layer, predict, target, T1, T2, t, lam, full_target):
    raise NotImplementedError("write your pallas kernel here")


# speedup vs baseline: 21.1574x; 21.1574x over previous
import numpy as np
import jax
import jax.numpy as jnp
from jax.experimental import pallas as pl
from jax.experimental.pallas import tpu as pltpu

_H = 512
_W = 512
_HW = _H * _W
_D = 64
_N = 2
_ROWS = 64
_K = _H // _ROWS
_GAMMA = 0.9


def _rank_maps(n: int) -> jnp.ndarray:
    rng = np.random.RandomState(0)
    out = np.empty((n, _HW), np.int32)
    for i in range(n):
        perm = rng.permutation(_HW)
        out[i, perm] = np.arange(_HW, dtype=np.int32)
    return jnp.asarray(out.reshape(n, _H, _W))


_RANKS = _rank_maps(_N)


def _stats_kernel(tgt_ref, ft_ref, rank_ref, activ_ref, out_ref, cnt_ref):
    k = pl.program_id(1)

    @pl.when(k == 0)
    def _():
        nfg = jnp.sum((tgt_ref[0] == 1).astype(jnp.float32))
        sel_full = (rank_ref[0].astype(jnp.float32) < nfg) & (ft_ref[0] == 0)
        cnt_ref[0] = nfg
        cnt_ref[1] = jnp.sum(sel_full.astype(jnp.float32))

    nfg = cnt_ref[0]
    r0 = k * _ROWS
    tgt = tgt_ref[0, pl.ds(r0, _ROWS), :]
    ft = ft_ref[0, pl.ds(r0, _ROWS), :]
    rank = rank_ref[0, pl.ds(r0, _ROWS), :]
    fg = (tgt == 1).astype(jnp.float32)
    sel = ((rank.astype(jnp.float32) < nfg)
           & (ft == 0)).astype(jnp.float32)
    a = activ_ref[0]
    a2 = a * a
    p_sel = jnp.sum(a * sel[None, :, :], axis=1)
    p_fg = jnp.sum(a * fg[None, :, :], axis=1)
    q_sel = jnp.sum(a2 * sel[None, :, :], axis=1)
    q_fg = jnp.sum(a2 * fg[None, :, :], axis=1)
    part = jnp.stack([p_sel, p_fg, q_sel, q_fg], axis=0)

    @pl.when(k == 0)
    def _():
        out_ref[0, 0:4] = part

    @pl.when(k > 0)
    def _():
        out_ref[0, 0:4] = out_ref[0, 0:4] + part

    @pl.when(k == _K - 1)
    def _():
        out_ref[0, 4] = jnp.full((_D, _W), cnt_ref[0], jnp.float32)
        out_ref[0, 5] = jnp.full((_D, _W), cnt_ref[1], jnp.float32)


def _loss_kernel(scal_ref, tgt_ref, ft_ref, rank_ref, params_ref,
                 activ_ref, pred_ref, out_ref, acc_ref):
    i = pl.program_id(0)
    k = pl.program_id(1)
    nfg = scal_ref[i, 0]
    nc0 = scal_ref[i, 1]
    nc1 = scal_ref[i, 2]

    @pl.when(k == 0)
    def _():
        acc_ref[0] = 0.0
        acc_ref[1] = 0.0
        acc_ref[2] = 0.0
        acc_ref[3] = 0.0

    r0 = k * _ROWS
    tgt = tgt_ref[0, pl.ds(r0, _ROWS), :]
    ft = ft_ref[0, pl.ds(r0, _ROWS), :]
    rank = rank_ref[0, pl.ds(r0, _ROWS), :]
    fg_b = tgt == 1
    sel_b = (rank.astype(jnp.float32) < nfg) & (ft == 0)

    a = activ_ref[0]
    c0 = jnp.reshape(params_ref[0, :, 0:1], (_D, 1, 1))
    c1 = jnp.reshape(params_ref[0, :, 1:2], (_D, 1, 1))
    dot0 = jnp.sum(a * c0, axis=0)
    dot1 = jnp.sum(a * c1, axis=0)
    norm_p = jnp.sqrt(jnp.sum(a * a, axis=0))
    sim0 = dot0 / (nc0 * norm_p)
    sim1 = dot1 / (nc1 * norm_p)
    pseudo = jnp.where(sim1 > _GAMMA, 1, jnp.where(sim0 > _GAMMA, 0, 2))

    w_valid = ((tgt == 0) & (~sel_b) & (pseudo != 2)).astype(jnp.float32)
    sup = (fg_b | sel_b).astype(jnp.float32)

    p0 = pred_ref[0, 0]
    p1 = pred_ref[0, 1]
    m = jnp.maximum(p0, p1)
    lse = m + jnp.log(jnp.exp(p0 - m) + jnp.exp(p1 - m))
    ce0 = lse - p0
    ce1 = lse - p1
    ce_sup = jnp.where(tgt == 1, ce1, ce0)
    ce_self = jnp.where(pseudo >= 1, ce1, ce0)

    acc_ref[0] += jnp.sum(ce_sup * sup)
    acc_ref[1] += jnp.sum(sup)
    acc_ref[2] += jnp.sum(ce_self * w_valid)
    acc_ref[3] += jnp.sum(w_valid)

    @pl.when(k == _K - 1)
    def _():
        out_ref[0, 0] = jnp.full((8, 128), acc_ref[0], jnp.float32)
        out_ref[0, 1] = jnp.full((8, 128), acc_ref[1], jnp.float32)
        out_ref[0, 2] = jnp.full((8, 128), acc_ref[2], jnp.float32)
        out_ref[0, 3] = jnp.full((8, 128), acc_ref[3], jnp.float32)


def _img_spec():
    return pl.BlockSpec((1, _H, _W), lambda i, k: (i, 0, 0))


def kernel(activ_last_layer, predict, target, T1, T2, t, lam, full_target):
    n, d, h, w = activ_last_layer.shape
    grid = (n, _K)

    stats = pl.pallas_call(
        _stats_kernel,
        grid=grid,
        in_specs=[
            _img_spec(),
            _img_spec(),
            _img_spec(),
            pl.BlockSpec((1, _D, _ROWS, _W), lambda i, k: (i, 0, k, 0)),
        ],
        out_specs=pl.BlockSpec((1, 6, _D, _W), lambda i, k: (i, 0, 0, 0)),
        out_shape=jax.ShapeDtypeStruct((n, 6, _D, _W), jnp.float32),
        scratch_shapes=[pltpu.SMEM((2,), jnp.float32)],
    )(target, full_target, _RANKS, activ_last_layer)

    vecs = jnp.sum(stats[:, :4], axis=-1)
    nfg = stats[:, 4, 0, 0]
    nsel = stats[:, 5, 0, 0]
    c0 = vecs[:, 0] / nsel[:, None]
    c1 = vecs[:, 1] / nfg[:, None]
    sq0 = vecs[:, 2]
    sq1 = vecs[:, 3]
    nc0 = jnp.sqrt(jnp.sum(c0 * c0, axis=1))
    nc1 = jnp.sqrt(jnp.sum(c1 * c1, axis=1))
    fisher_num = jnp.sum(jnp.sum(c0 * c1, axis=1) / (nc0 * nc1))
    den0 = jnp.sqrt(jnp.sum(c0 * c0 * sq0, axis=1))
    den1 = jnp.sqrt(jnp.sum(c1 * c1 * sq1, axis=1))
    fisher_den = jnp.sum(nsel * nc0 ** 2 / den0 + nfg * nc1 ** 2 / den1)

    params = jnp.zeros((n, _D, 128), jnp.float32)
    params = params.at[:, :, 0].set(c0).at[:, :, 1].set(c1)
    scal = jnp.stack([nfg, nc0, nc1], axis=1)

    sums = pl.pallas_call(
        _loss_kernel,
        grid=grid,
        in_specs=[
            pl.BlockSpec(memory_space=pltpu.SMEM),
            _img_spec(),
            _img_spec(),
            _img_spec(),
            pl.BlockSpec((1, _D, 128), lambda i, k: (i, 0, 0)),
            pl.BlockSpec((1, _D, _ROWS, _W), lambda i, k: (i, 0, k, 0)),
            pl.BlockSpec((1, 2, _ROWS, _W), lambda i, k: (i, 0, k, 0)),
        ],
        out_specs=pl.BlockSpec((1, 4, 8, 128), lambda i, k: (i, 0, 0, 0)),
        out_shape=jax.ShapeDtypeStruct((n, 4, 8, 128), jnp.float32),
        scratch_shapes=[pltpu.SMEM((4,), jnp.float32)],
    )(scal, target, full_target, _RANKS, params, activ_last_layer, predict)

    s = sums[:, :, 0, 0]
    loss_sup = jnp.sum(s[:, 0] / s[:, 1])
    loss_self_sup = jnp.sum(s[:, 2] / s[:, 3])
    loss_fisher = fisher_num / fisher_den

    beta = 0.0
    nan_flag = jnp.isnan(loss_self_sup) | jnp.isnan(loss_fisher)
    alpha = jnp.where(t < T1, jnp.float32(0.0),
                      jnp.where(t < T2, (t - T1) * lam / (T2 - T1), lam))
    loss = loss_sup + alpha * loss_self_sup + beta * loss_fisher
    out = jnp.where(nan_flag, loss_sup, loss)
    return jnp.where(t < T1, loss_sup, out)


# speedup vs baseline: 26.9881x; 1.2756x over previous
import numpy as np
import jax
import jax.numpy as jnp
from jax.experimental import pallas as pl
from jax.experimental.pallas import tpu as pltpu

_H = 512
_W = 512
_HW = _H * _W
_D = 64
_N = 2
_ROWS = 64
_K = _H // _ROWS
_GAMMA = 0.9


def _rank_maps(n: int) -> np.ndarray:
    rng = np.random.RandomState(0)
    out = np.empty((n, _HW), np.int32)
    for i in range(n):
        perm = rng.permutation(_HW)
        out[i, perm] = np.arange(_HW, dtype=np.int32)
    return out.reshape(n, _H, _W)


_RANKS = _rank_maps(_N)


def _stats_kernel(tgt_ref, ft_ref, rank_ref, activ_ref, out_ref, cnt_ref):
    k = pl.program_id(1)

    @pl.when(k == 0)
    def _():
        nfg = jnp.sum((tgt_ref[0] == 1).astype(jnp.float32))
        sel_full = (rank_ref[0].astype(jnp.float32) < nfg) & (ft_ref[0] == 0)
        cnt_ref[0] = nfg
        cnt_ref[1] = jnp.sum(sel_full.astype(jnp.float32))

    nfg = cnt_ref[0]
    r0 = k * _ROWS
    tgt = tgt_ref[0, pl.ds(r0, _ROWS), :]
    ft = ft_ref[0, pl.ds(r0, _ROWS), :]
    rank = rank_ref[0, pl.ds(r0, _ROWS), :]
    fg = (tgt == 1).astype(jnp.float32)
    sel = ((rank.astype(jnp.float32) < nfg)
           & (ft == 0)).astype(jnp.float32)
    a = activ_ref[0]
    p_sel = jnp.sum(a * sel[None, :, :], axis=1)
    p_fg = jnp.sum(a * fg[None, :, :], axis=1)
    part = jnp.stack([p_sel, p_fg], axis=0)

    @pl.when(k == 0)
    def _():
        out_ref[0, 0:2] = part

    @pl.when(k > 0)
    def _():
        out_ref[0, 0:2] = out_ref[0, 0:2] + part

    @pl.when(k == _K - 1)
    def _():
        out_ref[0, 2] = jnp.full((_D, _W), cnt_ref[0], jnp.float32)
        out_ref[0, 3] = jnp.full((_D, _W), cnt_ref[1], jnp.float32)


def _loss_kernel(scal_ref, tgt_ref, ft_ref, rank_ref, params_ref,
                 activ_ref, pred_ref, out_ref, acc_ref):
    i = pl.program_id(0)
    k = pl.program_id(1)
    nfg = scal_ref[i, 0]
    nc0 = scal_ref[i, 1]
    nc1 = scal_ref[i, 2]

    @pl.when(k == 0)
    def _():
        acc_ref[0] = 0.0
        acc_ref[1] = 0.0
        acc_ref[2] = 0.0
        acc_ref[3] = 0.0

    r0 = k * _ROWS
    tgt = tgt_ref[0, pl.ds(r0, _ROWS), :]
    ft = ft_ref[0, pl.ds(r0, _ROWS), :]
    rank = rank_ref[0, pl.ds(r0, _ROWS), :]
    fg_b = tgt == 1
    sel_b = (rank.astype(jnp.float32) < nfg) & (ft == 0)

    a = activ_ref[0]
    c0 = jnp.reshape(params_ref[0, :, 0:1], (_D, 1, 1))
    c1 = jnp.reshape(params_ref[0, :, 1:2], (_D, 1, 1))
    dot0 = jnp.sum(a * c0, axis=0)
    dot1 = jnp.sum(a * c1, axis=0)
    norm_p = jnp.sqrt(jnp.sum(a * a, axis=0))
    sim0 = dot0 / (nc0 * norm_p)
    sim1 = dot1 / (nc1 * norm_p)
    pseudo = jnp.where(sim1 > _GAMMA, 1, jnp.where(sim0 > _GAMMA, 0, 2))

    w_valid = ((tgt == 0) & (~sel_b) & (pseudo != 2)).astype(jnp.float32)
    sup = (fg_b | sel_b).astype(jnp.float32)

    p0 = pred_ref[0, 0]
    p1 = pred_ref[0, 1]
    m = jnp.maximum(p0, p1)
    lse = m + jnp.log(jnp.exp(p0 - m) + jnp.exp(p1 - m))
    ce0 = lse - p0
    ce1 = lse - p1
    ce_sup = jnp.where(tgt == 1, ce1, ce0)
    ce_self = jnp.where(pseudo >= 1, ce1, ce0)

    acc_ref[0] += jnp.sum(ce_sup * sup)
    acc_ref[1] += jnp.sum(sup)
    acc_ref[2] += jnp.sum(ce_self * w_valid)
    acc_ref[3] += jnp.sum(w_valid)

    @pl.when(k == _K - 1)
    def _():
        out_ref[0, 0] = jnp.full((8, 128), acc_ref[0], jnp.float32)
        out_ref[0, 1] = jnp.full((8, 128), acc_ref[1], jnp.float32)
        out_ref[0, 2] = jnp.full((8, 128), acc_ref[2], jnp.float32)
        out_ref[0, 3] = jnp.full((8, 128), acc_ref[3], jnp.float32)


def _img_spec():
    return pl.BlockSpec((1, _H, _W), lambda i, k: (i, 0, 0))


def kernel(activ_last_layer, predict, target, T1, T2, t, lam, full_target):
    n, d, h, w = activ_last_layer.shape
    grid = (n, _K)

    stats = pl.pallas_call(
        _stats_kernel,
        grid=grid,
        in_specs=[
            _img_spec(),
            _img_spec(),
            _img_spec(),
            pl.BlockSpec((1, _D, _ROWS, _W), lambda i, k: (i, 0, k, 0)),
        ],
        out_specs=pl.BlockSpec((1, 4, _D, _W), lambda i, k: (i, 0, 0, 0)),
        out_shape=jax.ShapeDtypeStruct((n, 4, _D, _W), jnp.float32),
        scratch_shapes=[pltpu.SMEM((2,), jnp.float32)],
    )(target, full_target, _RANKS, activ_last_layer)

    vecs = jnp.sum(stats[:, :2], axis=-1)
    nfg = stats[:, 2, 0, 0]
    nsel = stats[:, 3, 0, 0]
    c0 = vecs[:, 0] / nsel[:, None]
    c1 = vecs[:, 1] / nfg[:, None]
    nc0 = jnp.sqrt(jnp.sum(c0 * c0, axis=1))
    nc1 = jnp.sqrt(jnp.sum(c1 * c1, axis=1))
    fisher_num = jnp.sum(jnp.sum(c0 * c1, axis=1) / (nc0 * nc1))

    params = jnp.zeros((n, _D, 128), jnp.float32)
    params = params.at[:, :, 0].set(c0).at[:, :, 1].set(c1)
    scal = jnp.stack([nfg, nc0, nc1], axis=1)

    sums = pl.pallas_call(
        _loss_kernel,
        grid=grid,
        in_specs=[
            pl.BlockSpec(memory_space=pltpu.SMEM),
            _img_spec(),
            _img_spec(),
            _img_spec(),
            pl.BlockSpec((1, _D, 128), lambda i, k: (i, 0, 0)),
            pl.BlockSpec((1, _D, _ROWS, _W), lambda i, k: (i, 0, k, 0)),
            pl.BlockSpec((1, 2, _ROWS, _W), lambda i, k: (i, 0, k, 0)),
        ],
        out_specs=pl.BlockSpec((1, 4, 8, 128), lambda i, k: (i, 0, 0, 0)),
        out_shape=jax.ShapeDtypeStruct((n, 4, 8, 128), jnp.float32),
        scratch_shapes=[pltpu.SMEM((4,), jnp.float32)],
    )(scal, target, full_target, _RANKS, params, activ_last_layer, predict)

    s = sums[:, :, 0, 0]
    loss_sup = jnp.sum(s[:, 0] / s[:, 1])
    loss_self_sup = jnp.sum(s[:, 2] / s[:, 3])

    nan_flag = jnp.isnan(loss_self_sup) | jnp.isnan(fisher_num)
    alpha = jnp.where(t < T1, jnp.float32(0.0),
                      jnp.where(t < T2, (t - T1) * lam / (T2 - T1), lam))
    loss = loss_sup + alpha * loss_self_sup
    out = jnp.where(nan_flag, loss_sup, loss)
    return jnp.where(t < T1, loss_sup, out)
